# baseline (speedup 1.0000x reference)
import functools

import jax
import jax.numpy as jnp
from jax import lax
from jax.experimental import pallas as pl
from jax.experimental.pallas import tpu as pltpu
from jax.experimental.pallas import tpu_sc as plsc

NC = 2
NS = 16
L = 16
NW = NC * NS

CHUNK = 100


def _make_kernel(V, S, E, B):
  assert S == 2 * CHUNK
  N = B * S
  assert N % (NW * S) == 0
  SPW = N // (NW * S)
  mesh = plsc.VectorSubcoreMesh(
      core_axis_name="c", subcore_axis_name="s",
      num_cores=NC, num_subcores=NS)

  @functools.partial(
      pl.kernel,
      out_type=jax.ShapeDtypeStruct((N, E), jnp.float32),
      mesh=mesh,
      compiler_params=pltpu.CompilerParams(use_tc_tiling_on_sc=False),
      scratch_types=[
          pltpu.VMEM((2 * SPW, CHUNK), jnp.int32),
          pltpu.VMEM((S, E), jnp.float32),
          pltpu.VMEM((S, E), jnp.float32),
          pltpu.SemaphoreType.DMA,
      ],
  )
  def k(tab_hbm, pos_hbm, idx_hbm, out_hbm, idx_v, pos_v, rows_v, gsem):
    wid = lax.axis_index("s") * NC + lax.axis_index("c")
    pltpu.sync_copy(idx_hbm.at[wid], idx_v)
    pltpu.sync_copy(pos_hbm, pos_v)

    @pl.loop(0, SPW)
    def _seq(q):
      pltpu.async_copy(tab_hbm.at[idx_v.at[2 * q]],
                       rows_v.at[pl.ds(0, CHUNK)], gsem)
      pltpu.async_copy(tab_hbm.at[idx_v.at[2 * q + 1]],
                       rows_v.at[pl.ds(CHUNK, CHUNK)], gsem)
      pltpu.make_async_copy(tab_hbm.at[idx_v.at[2 * q]],
                            rows_v.at[pl.ds(0, CHUNK)], gsem).wait()
      pltpu.make_async_copy(tab_hbm.at[idx_v.at[2 * q + 1]],
                            rows_v.at[pl.ds(CHUNK, CHUNK)], gsem).wait()

      @pl.loop(0, S)
      def _row(r):
        for j in range(E // L):
          sl = pl.ds(j * L, L)
          rows_v[r, sl] = rows_v[r, sl] + pos_v[r, sl]

      pltpu.sync_copy(rows_v, out_hbm.at[pl.ds((wid * SPW + q) * S, S)])

  return k


def kernel(x, token_table, pos_table):
  B, S = x.shape
  V, E = token_table.shape
  idx = x.astype(jnp.int32).reshape(NW, (B * S) // (NW * CHUNK), CHUNK)
  assert S == 2 * CHUNK
  k = _make_kernel(V, S, E, B)
  out = k(token_table, pos_table, idx)
  return out.reshape(B, S, E)


# speedup vs baseline: 1.0091x; 1.0091x over previous
import functools

import jax
import jax.numpy as jnp
from jax import lax
from jax.experimental import pallas as pl
from jax.experimental.pallas import tpu as pltpu
from jax.experimental.pallas import tpu_sc as plsc

NC = 2
NS = 16
L = 16
NW = NC * NS

CHUNK = 100


def _make_kernel(V, S, E, B):
  assert S == 2 * CHUNK
  N = B * S
  assert N % (NW * S) == 0
  SPW = N // (NW * S)
  mesh = plsc.VectorSubcoreMesh(
      core_axis_name="c", subcore_axis_name="s",
      num_cores=NC, num_subcores=NS)

  NBUF = 4
  assert SPW % NBUF == 0

  @functools.partial(
      pl.kernel,
      out_type=jax.ShapeDtypeStruct((N, E), jnp.float32),
      mesh=mesh,
      compiler_params=pltpu.CompilerParams(use_tc_tiling_on_sc=False),
      scratch_types=[
          pltpu.VMEM((2 * SPW, CHUNK), jnp.int32),
          pltpu.VMEM((S, E), jnp.float32),
          pltpu.VMEM((NBUF, S, E), jnp.float32),
          [pltpu.SemaphoreType.DMA] * NBUF,
          [pltpu.SemaphoreType.DMA] * NBUF,
      ],
  )
  def k(tab_hbm, pos_hbm, idx_hbm, out_hbm, idx_v, pos_v, rows_v, gsems, ssems):
    wid = lax.axis_index("s") * NC + lax.axis_index("c")
    pltpu.sync_copy(idx_hbm.at[wid], idx_v)
    pltpu.sync_copy(pos_hbm, pos_v)

    def gather_start(q, b):
      pltpu.async_copy(tab_hbm.at[idx_v.at[2 * q]],
                       rows_v.at[b].at[pl.ds(0, CHUNK)], gsems[b])
      pltpu.async_copy(tab_hbm.at[idx_v.at[2 * q + 1]],
                       rows_v.at[b].at[pl.ds(CHUNK, CHUNK)], gsems[b])

    def gather_wait(q, b):
      pltpu.make_async_copy(tab_hbm.at[idx_v.at[2 * q]],
                            rows_v.at[b].at[pl.ds(0, CHUNK)], gsems[b]).wait()
      pltpu.make_async_copy(tab_hbm.at[idx_v.at[2 * q + 1]],
                            rows_v.at[b].at[pl.ds(CHUNK, CHUNK)],
                            gsems[b]).wait()

    def out_ref(q):
      return out_hbm.at[pl.ds((wid * SPW + q) * S, S)]

    def store_start(q, b):
      pltpu.async_copy(rows_v.at[b], out_ref(q), ssems[b])

    def store_wait(q, b):
      pltpu.make_async_copy(rows_v.at[b], out_ref(q), ssems[b]).wait()

    for j in range(NBUF - 1):
      gather_start(j, j)

    @pl.loop(0, SPW, step=NBUF)
    def _grp(g):
      for kk in range(NBUF):
        c = g + kk
        b = kk
        bp = (kk + NBUF - 1) % NBUF
        gather_wait(c, b)

        @pl.loop(0, S, unroll=8)
        def _row(r):
          for j in range(E // L):
            sl = pl.ds(j * L, L)
            rows_v[b, r, sl] = rows_v[b, r, sl] + pos_v[r, sl]

        store_start(c, b)
        if kk == 0:
          @pl.when(c >= 1)
          def _():
            store_wait(c - 1, bp)
        else:
          store_wait(c - 1, bp)

        @pl.when(c < SPW - (NBUF - 1))
        def _():
          gather_start(c + NBUF - 1, bp)

    store_wait(SPW - 1, NBUF - 1)

  return k


def kernel(x, token_table, pos_table):
  B, S = x.shape
  V, E = token_table.shape
  idx = x.astype(jnp.int32).reshape(NW, (B * S) // (NW * CHUNK), CHUNK)
  assert S == 2 * CHUNK
  k = _make_kernel(V, S, E, B)
  out = k(token_table, pos_table, idx)
  return out.reshape(B, S, E)
